# baseline (speedup 1.0000x reference)
import functools

import jax
import jax.numpy as jnp
from jax import lax
from jax.experimental import pallas as pl
from jax.experimental.pallas import tpu as pltpu
from jax.experimental.pallas import tpu_sc as plsc

LAMDA = 0.001

NC = 2
NS = 16
NW = NC * NS
L = 16

B = 16384
D = 64
BPW = B // NW
CH = 128
NCHUNK = BPW // CH


def _sc_body(u_h, i_h, pu_h, pi_h, nu_h, ni_h, r_h, nr_h, wu, wi, wr,
             xhat_o, sq_o,
             u_v, i_v, pu_v, pi_v, nu_v, ni_v, r_v, nr_v,
             ri_v, pri_v, nri_v,
             bu, bi, bpu, bpi, bnu, bni, br, bpr, bnr,
             xhat_v, squ, sqi, sqpu, sqpi, sqnu, sqni, sqr, sqnr, sem):
    user_num = wu.shape[0]
    wid = lax.axis_index("s") * NC + lax.axis_index("c")
    base = pl.multiple_of(wid * BPW, BPW)

    pltpu.sync_copy(u_h.at[pl.ds(base, BPW)], u_v)
    pltpu.sync_copy(i_h.at[pl.ds(base, BPW)], i_v)
    pltpu.sync_copy(pu_h.at[pl.ds(base, BPW)], pu_v)
    pltpu.sync_copy(pi_h.at[pl.ds(base, BPW)], pi_v)
    pltpu.sync_copy(nu_h.at[pl.ds(base, BPW)], nu_v)
    pltpu.sync_copy(ni_h.at[pl.ds(base, BPW)], ni_v)
    pltpu.sync_copy(r_h.at[pl.ds(base, BPW)], r_v)
    pltpu.sync_copy(nr_h.at[pl.ds(base, BPW)], nr_v)

    def idx_body(k, _):
        s = pl.ds(pl.multiple_of(k * L, L), L)
        rv = r_v[s]
        ri_v[s] = u_v[s] + rv * user_num
        pri_v[s] = pu_v[s] + rv * user_num
        nri_v[s] = nu_v[s] + nr_v[s] * user_num
        return 0

    lax.fori_loop(0, BPW // L, idx_body, 0)

    rows0 = lax.iota(jnp.int32, L)
    zero = jnp.zeros((L,), jnp.float32)

    def hsum(v):
        for sh in (8, 4, 2, 1):
            perm = jnp.bitwise_xor(rows0, sh)
            v = v + jnp.take_along_axis(v, perm, axis=0,
                                        mode="promise_in_bounds")
        return v

    for c in range(NCHUNK):
        s = pl.ds(c * CH, CH)
        cps = [
            pltpu.async_copy(wu.at[u_v.at[s]], bu, sem),
            pltpu.async_copy(wi.at[i_v.at[s]], bi, sem),
            pltpu.async_copy(wu.at[pu_v.at[s]], bpu, sem),
            pltpu.async_copy(wi.at[pi_v.at[s]], bpi, sem),
            pltpu.async_copy(wu.at[nu_v.at[s]], bnu, sem),
            pltpu.async_copy(wi.at[ni_v.at[s]], bni, sem),
            pltpu.async_copy(wr.at[ri_v.at[s]], br, sem),
            pltpu.async_copy(wr.at[pri_v.at[s]], bpr, sem),
            pltpu.async_copy(wr.at[nri_v.at[s]], bnr, sem),
        ]
        for cp in cps:
            cp.wait()

        def group_body(g, _, c=c):
            def row_body(rr, acc):
                xh, au, ai, apu, api, anu, ani, ar, anr = acc
                r = g * L + rr
                part = [zero] * 9
                for k in range(D // L):
                    sk = pl.ds(k * L, L)
                    uv = bu[r, sk]
                    iv = bi[r, sk]
                    puv = bpu[r, sk]
                    piv = bpi[r, sk]
                    nuv = bnu[r, sk]
                    niv = bni[r, sk]
                    rv = br[r, sk]
                    prv = bpr[r, sk]
                    nrv = bnr[r, sk]
                    gv = uv + rv + iv
                    gp = puv + prv + piv
                    gn = nuv + nrv + niv
                    part = [part[0] + gv * (gp - gn), part[1] + uv * uv,
                            part[2] + iv * iv, part[3] + puv * puv,
                            part[4] + piv * piv, part[5] + nuv * nuv,
                            part[6] + niv * niv, part[7] + rv * rv,
                            part[8] + nrv * nrv]
                lane = rows0 == rr
                xh = jnp.where(lane, hsum(part[0]), xh)
                au = jnp.where(lane, hsum(part[1]), au)
                ai = jnp.where(lane, hsum(part[2]), ai)
                apu = jnp.where(lane, hsum(part[3]), apu)
                api = jnp.where(lane, hsum(part[4]), api)
                anu = jnp.where(lane, hsum(part[5]), anu)
                ani = jnp.where(lane, hsum(part[6]), ani)
                ar = jnp.where(lane, hsum(part[7]), ar)
                anr = jnp.where(lane, hsum(part[8]), anr)
                return (xh, au, ai, apu, api, anu, ani, ar, anr)

            xh, au, ai, apu, api, anu, ani, ar, anr = lax.fori_loop(
                0, L, row_body, (zero,) * 9)
            so = pl.ds(pl.multiple_of(c * CH + g * L, L), L)
            xhat_v[so] = xh
            squ[so] = au
            sqi[so] = ai
            sqpu[so] = apu
            sqpi[so] = api
            sqnu[so] = anu
            sqni[so] = ani
            sqr[so] = ar
            sqnr[so] = anr
            return 0

        lax.fori_loop(0, CH // L, group_body, 0)

    pltpu.sync_copy(xhat_v, xhat_o.at[pl.ds(base, BPW)])
    pltpu.sync_copy(squ, sq_o.at[0, pl.ds(base, BPW)])
    pltpu.sync_copy(sqi, sq_o.at[1, pl.ds(base, BPW)])
    pltpu.sync_copy(sqpu, sq_o.at[2, pl.ds(base, BPW)])
    pltpu.sync_copy(sqpi, sq_o.at[3, pl.ds(base, BPW)])
    pltpu.sync_copy(sqnu, sq_o.at[4, pl.ds(base, BPW)])
    pltpu.sync_copy(sqni, sq_o.at[5, pl.ds(base, BPW)])
    pltpu.sync_copy(sqr, sq_o.at[6, pl.ds(base, BPW)])
    pltpu.sync_copy(sqnr, sq_o.at[7, pl.ds(base, BPW)])


_sc_call = functools.partial(
    pl.kernel,
    out_type=(jax.ShapeDtypeStruct((B,), jnp.float32),
              jax.ShapeDtypeStruct((8, B), jnp.float32)),
    mesh=plsc.VectorSubcoreMesh(core_axis_name="c", subcore_axis_name="s",
                                num_cores=NC, num_subcores=NS),
    scratch_types=(
        [pltpu.VMEM((BPW,), jnp.int32)] * 11
        + [pltpu.VMEM((CH, D), jnp.float32)] * 9
        + [pltpu.VMEM((BPW,), jnp.float32)] * 9
        + [pltpu.SemaphoreType.DMA]
    ),
    compiler_params=pltpu.CompilerParams(use_tc_tiling_on_sc=False),
)(_sc_body)


def _fin_body(x_ref, s_ref, loss_ref, reg_ref):
    x = x_ref[...]
    p = 1.0 / (1.0 + jnp.exp(-x))
    loss_ref[0, 0] = -jnp.sum(jnp.log(p))
    reg_ref[0, 0] = jnp.sum(jnp.sqrt(s_ref[...])) * LAMDA


_fin_call = pl.pallas_call(
    _fin_body,
    out_shape=(jax.ShapeDtypeStruct((1, 1), jnp.float32),
               jax.ShapeDtypeStruct((1, 1), jnp.float32)),
    out_specs=(pl.BlockSpec(memory_space=pltpu.SMEM),
               pl.BlockSpec(memory_space=pltpu.SMEM)),
)


def kernel(user_idx, item_idx, pos_user_idx, pos_item_idx, neg_user_idx,
           neg_item_idx, rel_idx, neg_rel_idx, W_user, W_item, W_rel):
    xhat, sq = _sc_call(user_idx.astype(jnp.int32), item_idx.astype(jnp.int32),
                        pos_user_idx.astype(jnp.int32),
                        pos_item_idx.astype(jnp.int32),
                        neg_user_idx.astype(jnp.int32),
                        neg_item_idx.astype(jnp.int32),
                        rel_idx.astype(jnp.int32),
                        neg_rel_idx.astype(jnp.int32),
                        W_user, W_item, W_rel)
    loss, reg = _fin_call(xhat.reshape(128, 128), sq.reshape(1024, 128))
    return (loss[0, 0], reg[0, 0])
